# baseline (speedup 1.0000x reference)
import functools

import jax
import jax.numpy as jnp
from jax import lax
from jax.experimental import pallas as pl
from jax.experimental.pallas import tpu as pltpu
from jax.experimental.pallas import tpu_sc as plsc

TOTAL_TOKENS = 32768
BATCH = 16
D_MODEL = 4096
LANES = 16
CHUNK = D_MODEL // LANES
N_SLICES = D_MODEL // LANES

_mesh = plsc.VectorSubcoreMesh(core_axis_name="c", subcore_axis_name="s")


@functools.partial(
    pl.kernel,
    mesh=_mesh,
    out_type=jax.ShapeDtypeStruct((BATCH * LANES, CHUNK), jnp.float32),
    scratch_types=[
        pltpu.VMEM((LANES,), jnp.int32),
        pltpu.VMEM((LANES, CHUNK), jnp.float32),
        pltpu.SemaphoreType.DMA,
    ],
    compiler_params=pltpu.CompilerParams(needs_layout_passes=False),
)
def _pooler(hs_hbm, lens_hbm, out_hbm, lens_v, row_v, sem):
    wid = lax.axis_index("s") * 2 + lax.axis_index("c")

    @pl.when(wid < BATCH)
    def _():
        r = wid
        pltpu.sync_copy(lens_hbm, lens_v)
        lens = lens_v[...]
        csum = jnp.cumsum(lens)
        iota = lax.iota(jnp.int32, LANES)
        base = jnp.sum(jnp.where(iota == r, csum - 1, 0))
        vidx = base * LANES + iota
        pltpu.async_copy(hs_hbm.at[vidx], row_v, sem).wait()

        def ssq_body(k, acc):
            x = row_v[k // LANES, pl.ds((k % LANES) * LANES, LANES)]
            return acc + x * x

        acc = lax.fori_loop(0, N_SLICES, ssq_body, jnp.zeros((LANES,), jnp.float32))
        ssv = jnp.full((LANES,), jnp.sum(acc))
        ssv = jnp.maximum(ssv, 1e-24)
        y = plsc.bitcast(0x5F3759DF - lax.shift_right_logical(
            plsc.bitcast(ssv, jnp.int32), 1), jnp.float32)
        for _unused in range(3):
            y = y * (1.5 - 0.5 * ssv * y * y)

        def scale_body(k, carry):
            i = k // LANES
            j = (k % LANES) * LANES
            row_v[i, pl.ds(j, LANES)] = row_v[i, pl.ds(j, LANES)] * y
            return carry

        lax.fori_loop(0, N_SLICES, scale_body, 0)
        pltpu.sync_copy(row_v, out_hbm.at[pl.ds(r * LANES, LANES)])


def kernel(hidden_states, extend_seq_lens):
    hs_virt = hidden_states.reshape(TOTAL_TOKENS * LANES, CHUNK)
    out = _pooler(hs_virt, extend_seq_lens)
    return out.reshape(BATCH, D_MODEL)


# speedup vs baseline: 27.0985x; 27.0985x over previous
import functools

import jax
import jax.numpy as jnp
from jax import lax
from jax.experimental import pallas as pl
from jax.experimental.pallas import tpu as pltpu
from jax.experimental.pallas import tpu_sc as plsc

TOTAL_TOKENS = 32768
BATCH = 16
D_MODEL = 4096
LANES = 16
N_SLICES = D_MODEL // LANES

_mesh = plsc.VectorSubcoreMesh(core_axis_name="c", subcore_axis_name="s")


@functools.partial(
    pl.kernel,
    mesh=_mesh,
    out_type=jax.ShapeDtypeStruct((BATCH, D_MODEL), jnp.float32),
    scratch_types=[
        pltpu.VMEM((LANES,), jnp.int32),
        pltpu.VMEM((1,), jnp.int32),
        pltpu.VMEM((1, D_MODEL), jnp.float32),
        pltpu.SemaphoreType.DMA,
    ],
    compiler_params=pltpu.CompilerParams(needs_layout_passes=False),
)
def _pooler(hs_hbm, lens_hbm, out_hbm, lens_v, idx1, row_v, sem):
    wid = lax.axis_index("s") * 2 + lax.axis_index("c")

    @pl.when(wid < BATCH)
    def _():
        r = wid
        pltpu.sync_copy(lens_hbm, lens_v)
        lens = lens_v[...]
        csum = jnp.cumsum(lens)
        iota = lax.iota(jnp.int32, LANES)
        plsc.store_scatter(idx1, [jnp.zeros((LANES,), jnp.int32)], csum - 1,
                           mask=iota == r)
        pltpu.async_copy(hs_hbm.at[idx1], row_v, sem).wait()

        def ssq_body(k, acc):
            x = row_v[0, pl.ds(k * LANES, LANES)]
            return acc + x * x

        acc = lax.fori_loop(0, N_SLICES, ssq_body, jnp.zeros((LANES,), jnp.float32))
        ssv = jnp.full((LANES,), jnp.sum(acc))
        ssv = jnp.maximum(ssv, 1e-24)
        y = plsc.bitcast(0x5F3759DF - lax.shift_right_logical(
            plsc.bitcast(ssv, jnp.int32), 1), jnp.float32)
        for _unused in range(3):
            y = y * (1.5 - 0.5 * ssv * y * y)

        def scale_body(k, carry):
            row_v[0, pl.ds(k * LANES, LANES)] = row_v[0, pl.ds(k * LANES, LANES)] * y
            return carry

        lax.fori_loop(0, N_SLICES, scale_body, 0)
        pltpu.sync_copy(row_v, out_hbm.at[pl.ds(r, 1)])


def kernel(hidden_states, extend_seq_lens):
    return _pooler(hidden_states, extend_seq_lens)


# speedup vs baseline: 28.2372x; 1.0420x over previous
import functools

import jax
import jax.numpy as jnp
from jax import lax
from jax.experimental import pallas as pl
from jax.experimental.pallas import tpu as pltpu
from jax.experimental.pallas import tpu_sc as plsc

TOTAL_TOKENS = 32768
BATCH = 16
D_MODEL = 4096
LANES = 16
HALF = D_MODEL // 2
N_SLICES = D_MODEL // LANES
N_HSLICES = HALF // LANES

_mesh = plsc.VectorSubcoreMesh(core_axis_name="c", subcore_axis_name="s")


@functools.partial(
    pl.kernel,
    mesh=_mesh,
    out_type=jax.ShapeDtypeStruct((BATCH, D_MODEL), jnp.float32),
    scratch_types=[
        pltpu.VMEM((LANES,), jnp.int32),
        pltpu.VMEM((1,), jnp.int32),
        pltpu.VMEM((1, D_MODEL), jnp.float32),
        pltpu.SemaphoreType.DMA,
    ],
    compiler_params=pltpu.CompilerParams(needs_layout_passes=False),
)
def _pooler(hs_hbm, lens_hbm, out_hbm, lens_v, idx1, row_v, sem):
    c = lax.axis_index("c")
    s = lax.axis_index("s")
    r = c * 8 + s // 2
    h = s % 2

    pltpu.sync_copy(lens_hbm, lens_v)
    lens = lens_v[...]
    csum = jnp.cumsum(lens)
    iota = lax.iota(jnp.int32, LANES)
    plsc.store_scatter(idx1, [jnp.zeros((LANES,), jnp.int32)], csum - 1,
                       mask=iota == r)
    pltpu.async_copy(hs_hbm.at[idx1], row_v, sem).wait()

    @functools.partial(
        plsc.parallel_loop(0, N_SLICES, unroll=8,
                           carry=jnp.zeros((LANES,), jnp.float32))
    )
    def acc(k, a):
        x = row_v[0, pl.ds(k * LANES, LANES)]
        return a + x * x

    ssv = jnp.full((LANES,), jnp.sum(acc))
    ssv = jnp.maximum(ssv, 1e-24)
    y = plsc.bitcast(0x5F3759DF - lax.shift_right_logical(
        plsc.bitcast(ssv, jnp.int32), 1), jnp.float32)
    for _unused in range(3):
        y = y * (1.5 - 0.5 * ssv * y * y)

    @functools.partial(plsc.parallel_loop(0, N_HSLICES, unroll=8))
    def _scale(k):
        j = h * HALF + k * LANES
        row_v[0, pl.ds(j, LANES)] = row_v[0, pl.ds(j, LANES)] * y

    pltpu.sync_copy(row_v.at[pl.ds(0, 1), pl.ds(h * HALF, HALF)],
                    out_hbm.at[pl.ds(r, 1), pl.ds(h * HALF, HALF)])


def kernel(hidden_states, extend_seq_lens):
    return _pooler(hidden_states, extend_seq_lens)


# speedup vs baseline: 31.1342x; 1.1026x over previous
import functools

import jax
import jax.numpy as jnp
from jax import lax
from jax.experimental import pallas as pl
from jax.experimental.pallas import tpu as pltpu
from jax.experimental.pallas import tpu_sc as plsc

TOTAL_TOKENS = 32768
BATCH = 16
D_MODEL = 4096
LANES = 16
N_SLICES = D_MODEL // LANES

_mesh = plsc.VectorSubcoreMesh(core_axis_name="c", subcore_axis_name="s",
                               num_cores=1)


@functools.partial(
    pl.kernel,
    mesh=_mesh,
    out_type=jax.ShapeDtypeStruct((BATCH, D_MODEL), jnp.float32),
    scratch_types=[
        pltpu.VMEM((LANES,), jnp.int32),
        pltpu.VMEM((1,), jnp.int32),
        pltpu.VMEM((1, D_MODEL), jnp.float32),
        pltpu.SemaphoreType.DMA,
    ],
    compiler_params=pltpu.CompilerParams(needs_layout_passes=False),
)
def _pooler(hs_hbm, lens_hbm, out_hbm, lens_v, idx1, row_v, sem):
    r = lax.axis_index("s")

    pltpu.sync_copy(lens_hbm, lens_v)
    lens = lens_v[...]
    csum = jnp.cumsum(lens)
    iota = lax.iota(jnp.int32, LANES)
    plsc.store_scatter(idx1, [jnp.zeros((LANES,), jnp.int32)], csum - 1,
                       mask=iota == r)
    pltpu.async_copy(hs_hbm.at[idx1], row_v, sem).wait()

    @functools.partial(
        plsc.parallel_loop(0, N_SLICES, unroll=8,
                           carry=jnp.zeros((LANES,), jnp.float32))
    )
    def acc(k, a):
        x = row_v[0, pl.ds(k * LANES, LANES)]
        return a + x * x

    ssv = jnp.full((LANES,), jnp.sum(acc))
    ssv = jnp.maximum(ssv, 1e-24)
    y = plsc.bitcast(0x5F3759DF - lax.shift_right_logical(
        plsc.bitcast(ssv, jnp.int32), 1), jnp.float32)
    for _unused in range(3):
        y = y * (1.5 - 0.5 * ssv * y * y)

    @functools.partial(plsc.parallel_loop(0, N_SLICES, unroll=8))
    def _scale(k):
        row_v[0, pl.ds(k * LANES, LANES)] = row_v[0, pl.ds(k * LANES, LANES)] * y

    pltpu.sync_copy(row_v, out_hbm.at[pl.ds(r, 1)])


def kernel(hidden_states, extend_seq_lens):
    return _pooler(hidden_states, extend_seq_lens)
